# baseline (speedup 1.0000x reference)
import functools
import jax
import jax.numpy as jnp
from jax import lax
from jax.experimental import pallas as pl
from jax.experimental.pallas import tpu as pltpu
from jax.experimental.pallas import tpu_sc as plsc

_NC = 2
_NS = 16
_NW = _NC * _NS
_L = 16
_BB = 256


def _body(T, B0, H, ttab_hbm, idxt_hbm, out_hbm,
          ttile, idx0, idx1, stag0, stag1,
          sem_t, sem_i0, sem_i1, sem_w0, sem_w1):
    nblk = B0 // _BB
    nunits = T * nblk // (_NW // 2)
    wid = lax.axis_index("s") * _NC + lax.axis_index("c")
    ig = wid // 2
    h = wid % 2
    idxb = (idx0, idx1)
    stag = (stag0, stag1)
    sem_i = (sem_i0, sem_i1)
    sem_w = (sem_w0, sem_w1)

    pltpu.async_copy(ttab_hbm.at[pl.ds(h * H, H)], ttile, sem_t).wait()

    def unit_coords(u):
        g = ig + u * (_NW // 2)
        return g // nblk, g % nblk

    def idx_start(u, b):
        t, blk = unit_coords(u)
        pltpu.async_copy(
            idxt_hbm.at[t, pl.ds(blk * (_BB // _L), _BB // _L)],
            idxb[b], sem_i[b])

    def idx_wait(u, b):
        t, blk = unit_coords(u)
        pltpu.make_async_copy(
            idxt_hbm.at[t, pl.ds(blk * (_BB // _L), _BB // _L)],
            idxb[b], sem_i[b]).wait()

    def w_start(u, b):
        t, blk = unit_coords(u)
        pltpu.async_copy(
            stag[b], out_hbm.at[t, pl.ds(h * H, H), pl.ds(blk * _BB, _BB)],
            sem_w[b])

    def w_wait(u, b):
        t, blk = unit_coords(u)
        pltpu.make_async_copy(
            stag[b], out_hbm.at[t, pl.ds(h * H, H), pl.ds(blk * _BB, _BB)],
            sem_w[b]).wait()

    cols = [jnp.full((_L,), d, jnp.int32) for d in range(H)]

    def compute(b):
        @plsc.parallel_loop(0, _BB // _L, unroll=1)
        def group(g):
            iv = idxb[b][g]
            for d in range(H):
                vals = plsc.load_gather(ttile, [cols[d], iv])
                stag[b][d, pl.ds(g * _L, _L)] = vals

    for b in range(2):
        idx_start(b, b)

    def pair(p, carry):
        for b in range(2):
            u = 2 * p + b
            idx_wait(u, b)
            compute(b)
            w_start(u, b)
            w_wait(u, b)
            idx_start(u + 2, b)
        return carry

    npairs = nunits // 2
    lax.fori_loop(0, npairs - 1, pair, 0)

    for b in range(2):
        u = (npairs - 1) * 2 + b
        idx_wait(u, b)
        compute(b)
        w_start(u, b)
    for b in range(2):
        u = (npairs - 1) * 2 + b
        w_wait(u, b)


def kernel(input_pos_tensors, table):
    B0, T = input_pos_tensors.shape
    V, D = table.shape
    H = D // 2
    ttab = table.T
    idxt = input_pos_tensors.astype(jnp.int32).T.reshape(T, B0 // _L, _L)

    mesh = plsc.VectorSubcoreMesh(
        core_axis_name="c", subcore_axis_name="s",
        num_cores=_NC, num_subcores=_NS)
    run = pl.kernel(
        functools.partial(_body, T, B0, H),
        out_type=jax.ShapeDtypeStruct((T, D, B0), jnp.float32),
        mesh=mesh,
        scratch_types=[
            pltpu.VMEM((H, V), jnp.float32),
            pltpu.VMEM((_BB // _L, _L), jnp.int32),
            pltpu.VMEM((_BB // _L, _L), jnp.int32),
            pltpu.VMEM((H, _BB), jnp.float32),
            pltpu.VMEM((H, _BB), jnp.float32),
            pltpu.SemaphoreType.DMA,
            pltpu.SemaphoreType.DMA,
            pltpu.SemaphoreType.DMA,
            pltpu.SemaphoreType.DMA,
            pltpu.SemaphoreType.DMA,
        ],
        compiler_params=pltpu.CompilerParams(
            use_tc_tiling_on_sc=False, needs_layout_passes=False),
    )
    out = run(ttab, idxt)
    return jnp.transpose(out, (2, 0, 1))


# speedup vs baseline: 1.0447x; 1.0447x over previous
import functools
import jax
import jax.numpy as jnp
from jax import lax
from jax.experimental import pallas as pl
from jax.experimental.pallas import tpu as pltpu
from jax.experimental.pallas import tpu_sc as plsc

_NC = 2
_NS = 16
_NW = _NC * _NS
_L = 16
_BB = 512


def _body(T, B0, H, ttab_hbm, idxt_hbm, out_hbm,
          ttile, idx0, idx1, stag0, stag1,
          sem_t, sem_i0, sem_i1, sem_w0, sem_w1):
    nblk = B0 // _BB
    nunits = T * nblk // (_NW // 2)
    wid = lax.axis_index("s") * _NC + lax.axis_index("c")
    ig = wid // 2
    h = wid % 2
    idxb = (idx0, idx1)
    stag = (stag0, stag1)
    sem_i = (sem_i0, sem_i1)
    sem_w = (sem_w0, sem_w1)

    pltpu.async_copy(ttab_hbm.at[pl.ds(h * H, H)], ttile, sem_t).wait()

    def unit_coords(u):
        g = ig + u * (_NW // 2)
        return g // nblk, g % nblk

    def idx_start(u, b):
        t, blk = unit_coords(u)
        pltpu.async_copy(
            idxt_hbm.at[t, pl.ds(blk * (_BB // _L), _BB // _L)],
            idxb[b], sem_i[b])

    def idx_wait(u, b):
        t, blk = unit_coords(u)
        pltpu.make_async_copy(
            idxt_hbm.at[t, pl.ds(blk * (_BB // _L), _BB // _L)],
            idxb[b], sem_i[b]).wait()

    def w_start(u, b):
        t, blk = unit_coords(u)
        pltpu.async_copy(
            stag[b], out_hbm.at[t, pl.ds(h * H, H), pl.ds(blk * _BB, _BB)],
            sem_w[b])

    def w_wait(u, b):
        t, blk = unit_coords(u)
        pltpu.make_async_copy(
            stag[b], out_hbm.at[t, pl.ds(h * H, H), pl.ds(blk * _BB, _BB)],
            sem_w[b]).wait()

    cols = [jnp.full((_L,), d, jnp.int32) for d in range(H)]

    def compute(b):
        @plsc.parallel_loop(0, _BB // _L, unroll=1)
        def group(g):
            iv = idxb[b][g]
            for d in range(H):
                vals = plsc.load_gather(ttile, [cols[d], iv])
                stag[b][d, pl.ds(g * _L, _L)] = vals

    for b in range(2):
        idx_start(b, b)

    def pair(p, carry):
        for b in range(2):
            u = 2 * p + b
            idx_wait(u, b)
            compute(b)
            w_start(u, b)
            w_wait(u, b)
            idx_start(u + 2, b)
        return carry

    npairs = nunits // 2
    lax.fori_loop(0, npairs - 1, pair, 0)

    for b in range(2):
        u = (npairs - 1) * 2 + b
        idx_wait(u, b)
        compute(b)
        w_start(u, b)
    for b in range(2):
        u = (npairs - 1) * 2 + b
        w_wait(u, b)


def kernel(input_pos_tensors, table):
    B0, T = input_pos_tensors.shape
    V, D = table.shape
    H = D // 2
    ttab = table.T
    idxt = input_pos_tensors.astype(jnp.int32).T.reshape(T, B0 // _L, _L)

    mesh = plsc.VectorSubcoreMesh(
        core_axis_name="c", subcore_axis_name="s",
        num_cores=_NC, num_subcores=_NS)
    run = pl.kernel(
        functools.partial(_body, T, B0, H),
        out_type=jax.ShapeDtypeStruct((T, D, B0), jnp.float32),
        mesh=mesh,
        scratch_types=[
            pltpu.VMEM((H, V), jnp.float32),
            pltpu.VMEM((_BB // _L, _L), jnp.int32),
            pltpu.VMEM((_BB // _L, _L), jnp.int32),
            pltpu.VMEM((H, _BB), jnp.float32),
            pltpu.VMEM((H, _BB), jnp.float32),
            pltpu.SemaphoreType.DMA,
            pltpu.SemaphoreType.DMA,
            pltpu.SemaphoreType.DMA,
            pltpu.SemaphoreType.DMA,
            pltpu.SemaphoreType.DMA,
        ],
        compiler_params=pltpu.CompilerParams(
            use_tc_tiling_on_sc=False, needs_layout_passes=False),
    )
    out = run(ttab, idxt)
    return jnp.transpose(out, (2, 0, 1))
